# baseline (speedup 1.0000x reference)
import dataclasses
import functools

import jax
import jax.numpy as jnp
from jax import lax
from jax.experimental import pallas as pl
from jax.experimental.pallas import tpu as pltpu
from jax.experimental.pallas import tpu_sc as plsc

NC = 2
NS = 16
NW = NC * NS
LANES = 16
NBUF = 4


def _sc_compiler_params(tc_tiling=True):
    cp = pltpu.CompilerParams()
    if "needs_layout_passes" in pltpu.CompilerParams.__dataclass_fields__:
        cp = dataclasses.replace(cp, needs_layout_passes=False)
    if not tc_tiling:
        cp = dataclasses.replace(cp, use_tc_tiling_on_sc=False)
    return cp


def _sc_degree(dst, zeros_hist, n_pad):
    ep = dst.shape[0]
    e_per_tile = ep // NW
    mesh = plsc.VectorSubcoreMesh(core_axis_name="c", subcore_axis_name="s")

    @functools.partial(
        pl.kernel,
        out_type=jax.ShapeDtypeStruct((NW, n_pad), jnp.float32),
        mesh=mesh,
        scratch_types=[
            pltpu.VMEM((e_per_tile,), jnp.int32),
            pltpu.VMEM((n_pad,), jnp.float32),
        ],
        compiler_params=_sc_compiler_params(),
    )
    def deg_kernel(dst_hbm, zeros_hbm, out_hbm, dst_v, hist):
        c = lax.axis_index("c")
        s = lax.axis_index("s")
        wid = c * NS + s
        pltpu.sync_copy(dst_hbm.at[pl.ds(wid * e_per_tile, e_per_tile)], dst_v)
        pltpu.sync_copy(zeros_hbm, hist)
        ones = jnp.ones((LANES,), jnp.float32)

        @pl.loop(0, e_per_tile // LANES)
        def _(i):
            d = dst_v[pl.ds(i * LANES, LANES)]
            plsc.addupdate_scatter(hist, [d], ones)

        pltpu.sync_copy(hist, out_hbm.at[wid])

    return deg_kernel(dst, zeros_hist)


def _tc_transform(x_pad, W, deg_parts):
    n_pad, din = x_pad.shape
    dout = W.shape[1]
    hd = dout // NC
    blk = 1280
    grid = n_pad // blk

    def body(x_ref, w_ref, deg_ref, h2_ref, dinv_ref):
        deg = jnp.sum(deg_ref[...], axis=0) + 1.0
        dinv = lax.rsqrt(deg)
        h = jnp.dot(x_ref[...], w_ref[...], preferred_element_type=jnp.float32)
        h2 = h * dinv[:, None]
        for c in range(NC):
            h2_ref[c] = h2[:, c * hd:(c + 1) * hd]
        dinv_ref[pl.ds(pl.program_id(0) * blk, blk)] = dinv

    return pl.pallas_call(
        body,
        grid=(grid,),
        in_specs=[
            pl.BlockSpec((blk, din), lambda i: (i, 0)),
            pl.BlockSpec((din, dout), lambda i: (0, 0)),
            pl.BlockSpec((NW, blk), lambda i: (0, i)),
        ],
        out_specs=[
            pl.BlockSpec((NC, blk, hd), lambda i: (0, i, 0)),
            pl.BlockSpec((n_pad,), lambda i: (0,)),
        ],
        out_shape=[
            jax.ShapeDtypeStruct((NC, n_pad, hd), jnp.float32),
            jax.ShapeDtypeStruct((n_pad,), jnp.float32),
        ],
    )(x_pad, W, deg_parts)


def _sc_messages(src_t, dst_t, h2h, zeros_tile):
    _, ch, ck = src_t.shape
    _, n_pad, hd = h2h.shape
    rows_per_tile = n_pad // NS
    n_zero_copies = rows_per_tile // ck
    rounds = ch // NBUF
    mesh = plsc.VectorSubcoreMesh(core_axis_name="c", subcore_axis_name="s")

    @functools.partial(
        pl.kernel,
        out_type=jax.ShapeDtypeStruct((NC, n_pad, hd), jnp.float32),
        mesh=mesh,
        scratch_types=(
            [pltpu.VMEM((ch, ck), jnp.int32),
             pltpu.VMEM((ch, ck), jnp.int32),
             pltpu.VMEM_SHARED((n_pad, hd), jnp.float32)]
            + [pltpu.VMEM((ck, hd), jnp.float32) for _ in range(NBUF)]
            + [pltpu.SemaphoreType.DMA for _ in range(2 * NBUF)]
        ),
        compiler_params=_sc_compiler_params(tc_tiling=False),
    )
    def msg_kernel(src_hbm, dst_hbm, h2_hbm, z_hbm, out_hbm,
                   src_v, dst_v, acc, *bufs_and_sems):
        bufs = bufs_and_sems[:NBUF]
        gsems = bufs_and_sems[NBUF:2 * NBUF]
        ssems = bufs_and_sems[2 * NBUF:3 * NBUF]
        c = lax.axis_index("c")
        s = lax.axis_index("s")
        pltpu.sync_copy(src_hbm.at[s], src_v)
        pltpu.sync_copy(dst_hbm.at[s], dst_v)
        pltpu.sync_copy(z_hbm, bufs[0])
        for k in range(n_zero_copies):
            pltpu.sync_copy(bufs[0], acc.at[pl.ds(s * rows_per_tile + k * ck, ck)])
        plsc.subcore_barrier()

        def start_gather(j, b):
            pltpu.async_copy(h2_hbm.at[c].at[src_v.at[j]], bufs[b], gsems[b])

        def wait_gather(j, b):
            pltpu.make_async_copy(h2_hbm.at[c].at[src_v.at[j]], bufs[b],
                                  gsems[b]).wait()

        def start_scatter(j, b):
            pltpu.async_copy(bufs[b], acc.at[dst_v.at[j]], ssems[b], add=True)

        def wait_scatter(j, b):
            pltpu.make_async_copy(bufs[b], acc.at[dst_v.at[j]],
                                  ssems[b]).wait()

        for b in range(NBUF):
            start_gather(b, b)

        @pl.loop(0, rounds - 1)
        def _(jj):
            j0 = jj * NBUF
            for b in range(NBUF):
                wait_gather(j0 + b, b)
                start_scatter(j0 + b, b)
            for b in range(NBUF):
                wait_scatter(j0 + b, b)
                start_gather(j0 + NBUF + b, b)

        j0 = (rounds - 1) * NBUF
        for b in range(NBUF):
            wait_gather(j0 + b, b)
            start_scatter(j0 + b, b)
        for b in range(NBUF):
            wait_scatter(j0 + b, b)

        plsc.subcore_barrier()
        pltpu.sync_copy(acc.at[pl.ds(s * rows_per_tile, rows_per_tile)],
                        out_hbm.at[c, pl.ds(s * rows_per_tile, rows_per_tile)])

    return msg_kernel(src_t, dst_t, h2h, zeros_tile)


def _tc_epilogue(parts, h2h, dinv, b):
    _, n_pad, hd = h2h.shape
    dout = NC * hd
    blk = 2048
    grid = n_pad // blk

    def body(p_ref, h2_ref, dinv_ref, b_ref, o_ref):
        tot = jnp.concatenate(
            [p_ref[c] + h2_ref[c] for c in range(NC)], axis=1)
        dinv = dinv_ref[pl.ds(pl.program_id(0) * blk, blk)]
        o_ref[...] = jnp.maximum(
            tot * dinv[:, None] + b_ref[...][None, :], 0.0)

    return pl.pallas_call(
        body,
        grid=(grid,),
        in_specs=[
            pl.BlockSpec((NC, blk, hd), lambda i: (0, i, 0)),
            pl.BlockSpec((NC, blk, hd), lambda i: (0, i, 0)),
            pl.BlockSpec((n_pad,), lambda i: (0,)),
            pl.BlockSpec((dout,), lambda i: (0,)),
        ],
        out_specs=pl.BlockSpec((blk, dout), lambda i: (i, 0)),
        out_shape=jax.ShapeDtypeStruct((n_pad, dout), jnp.float32),
    )(parts, h2h, dinv, b)


def kernel(x, edge_index, W, b):
    n, din = x.shape
    dout = W.shape[1]
    hd = dout // NC
    e = edge_index.shape[1]
    src = edge_index[0].astype(jnp.int32)
    dst = edge_index[1].astype(jnp.int32)

    ck = 128
    n_pad = ((n + NS * ck - 1) // (NS * ck)) * (NS * ck)
    x_pad = jnp.concatenate(
        [x, jnp.zeros((n_pad - n, din), jnp.float32)], axis=0)

    e_deg = ((e + NW * LANES - 1) // (NW * LANES)) * (NW * LANES)
    dst_deg = jnp.concatenate(
        [dst, jnp.full((e_deg - e,), n, jnp.int32)])
    zeros_hist = jnp.zeros((n_pad,), jnp.float32)
    deg_parts = _sc_degree(dst_deg, zeros_hist, n_pad)

    h2h, dinv = _tc_transform(x_pad, W, deg_parts)

    e_pad = ((e + NS * ck * NBUF - 1) // (NS * ck * NBUF)) * (NS * ck * NBUF)
    pad = jnp.full((e_pad - e,), n, jnp.int32)
    src_t = jnp.concatenate([src, pad]).reshape(NS, e_pad // (NS * ck), ck)
    dst_t = jnp.concatenate([dst, pad]).reshape(NS, e_pad // (NS * ck), ck)
    zeros_tile = jnp.zeros((ck, hd), jnp.float32)
    parts = _sc_messages(src_t, dst_t, h2h, zeros_tile)

    return _tc_epilogue(parts, h2h, dinv, b)[:n]


# speedup vs baseline: 1.1081x; 1.1081x over previous
import dataclasses
import functools

import jax
import jax.numpy as jnp
from jax import lax
from jax.experimental import pallas as pl
from jax.experimental.pallas import tpu as pltpu
from jax.experimental.pallas import tpu_sc as plsc

NC = 2
NS = 16
NW = NC * NS
LANES = 16
NBUF = 4


def _sc_compiler_params(tc_tiling=True):
    cp = pltpu.CompilerParams()
    if "needs_layout_passes" in pltpu.CompilerParams.__dataclass_fields__:
        cp = dataclasses.replace(cp, needs_layout_passes=False)
    if not tc_tiling:
        cp = dataclasses.replace(cp, use_tc_tiling_on_sc=False)
    return cp


def _sc_degree(dst, zeros_hist, n_pad):
    ep = dst.shape[0]
    e_per_tile = ep // NW
    mesh = plsc.VectorSubcoreMesh(core_axis_name="c", subcore_axis_name="s")

    @functools.partial(
        pl.kernel,
        out_type=jax.ShapeDtypeStruct((NW, n_pad), jnp.float32),
        mesh=mesh,
        scratch_types=[
            pltpu.VMEM((e_per_tile,), jnp.int32),
            pltpu.VMEM((n_pad,), jnp.float32),
        ],
        compiler_params=_sc_compiler_params(),
    )
    def deg_kernel(dst_hbm, zeros_hbm, out_hbm, dst_v, hist):
        c = lax.axis_index("c")
        s = lax.axis_index("s")
        wid = c * NS + s
        pltpu.sync_copy(dst_hbm.at[pl.ds(wid * e_per_tile, e_per_tile)], dst_v)
        pltpu.sync_copy(zeros_hbm, hist)
        ones = jnp.ones((LANES,), jnp.float32)

        @pl.loop(0, e_per_tile // LANES)
        def _(i):
            d = dst_v[pl.ds(i * LANES, LANES)]
            plsc.addupdate_scatter(hist, [d], ones)

        pltpu.sync_copy(hist, out_hbm.at[wid])

    return deg_kernel(dst, zeros_hist)


def _tc_transform(x_pad, W, deg_parts):
    n_pad, din = x_pad.shape
    dout = W.shape[1]
    blk = 1280
    grid = n_pad // blk

    def body(x_ref, w_ref, deg_ref, h2_ref, h2b_ref, dinv_ref):
        deg = jnp.sum(deg_ref[...], axis=0) + 1.0
        dinv = lax.rsqrt(deg)
        h = jnp.dot(x_ref[...], w_ref[...], preferred_element_type=jnp.float32)
        h2 = h * dinv[:, None]
        h2_ref[...] = h2
        h2b_ref[...] = h2.astype(jnp.bfloat16)
        dinv_ref[pl.ds(pl.program_id(0) * blk, blk)] = dinv

    return pl.pallas_call(
        body,
        grid=(grid,),
        in_specs=[
            pl.BlockSpec((blk, din), lambda i: (i, 0)),
            pl.BlockSpec((din, dout), lambda i: (0, 0)),
            pl.BlockSpec((NW, blk), lambda i: (0, i)),
        ],
        out_specs=[
            pl.BlockSpec((blk, dout), lambda i: (i, 0)),
            pl.BlockSpec((blk, dout), lambda i: (i, 0)),
            pl.BlockSpec((n_pad,), lambda i: (0,)),
        ],
        out_shape=[
            jax.ShapeDtypeStruct((n_pad, dout), jnp.float32),
            jax.ShapeDtypeStruct((n_pad, dout), jnp.bfloat16),
            jax.ShapeDtypeStruct((n_pad,), jnp.float32),
        ],
    )(x_pad, W, deg_parts)


def _sc_messages(src_t, dst_t, h2b, zeros_tile):
    _, ch, ck = src_t.shape
    n_pad, dout = h2b.shape
    rows_per_tile = n_pad // NS
    n_zero_copies = rows_per_tile // ck
    rounds = ch // NBUF
    mesh = plsc.VectorSubcoreMesh(core_axis_name="c", subcore_axis_name="s")

    @functools.partial(
        pl.kernel,
        out_type=jax.ShapeDtypeStruct((NC, n_pad, dout), jnp.bfloat16),
        mesh=mesh,
        scratch_types=(
            [pltpu.VMEM((ch, ck), jnp.int32),
             pltpu.VMEM((ch, ck), jnp.int32),
             pltpu.VMEM_SHARED((n_pad, dout), jnp.bfloat16)]
            + [pltpu.VMEM((ck, dout), jnp.bfloat16) for _ in range(NBUF)]
            + [pltpu.SemaphoreType.DMA for _ in range(2 * NBUF)]
        ),
        compiler_params=_sc_compiler_params(tc_tiling=False),
    )
    def msg_kernel(src_hbm, dst_hbm, h2_hbm, z_hbm, out_hbm,
                   src_v, dst_v, acc, *bufs_and_sems):
        bufs = bufs_and_sems[:NBUF]
        gsems = bufs_and_sems[NBUF:2 * NBUF]
        ssems = bufs_and_sems[2 * NBUF:3 * NBUF]
        c = lax.axis_index("c")
        s = lax.axis_index("s")
        wid = c * NS + s
        pltpu.sync_copy(src_hbm.at[wid], src_v)
        pltpu.sync_copy(dst_hbm.at[wid], dst_v)
        pltpu.sync_copy(z_hbm, bufs[0])
        for k in range(n_zero_copies):
            pltpu.sync_copy(bufs[0], acc.at[pl.ds(s * rows_per_tile + k * ck, ck)])
        plsc.subcore_barrier()

        def start_gather(j, b):
            pltpu.async_copy(h2_hbm.at[src_v.at[j]], bufs[b], gsems[b])

        def wait_gather(j, b):
            pltpu.make_async_copy(h2_hbm.at[src_v.at[j]], bufs[b],
                                  gsems[b]).wait()

        def start_scatter(j, b):
            pltpu.async_copy(bufs[b], acc.at[dst_v.at[j]], ssems[b], add=True)

        def wait_scatter(j, b):
            pltpu.make_async_copy(bufs[b], acc.at[dst_v.at[j]],
                                  ssems[b]).wait()

        for b in range(NBUF):
            start_gather(b, b)

        @pl.loop(0, rounds - 1)
        def _(jj):
            j0 = jj * NBUF
            for b in range(NBUF):
                wait_gather(j0 + b, b)
                start_scatter(j0 + b, b)
            for b in range(NBUF):
                wait_scatter(j0 + b, b)
                start_gather(j0 + NBUF + b, b)

        j0 = (rounds - 1) * NBUF
        for b in range(NBUF):
            wait_gather(j0 + b, b)
            start_scatter(j0 + b, b)
        for b in range(NBUF):
            wait_scatter(j0 + b, b)

        plsc.subcore_barrier()
        pltpu.sync_copy(acc.at[pl.ds(s * rows_per_tile, rows_per_tile)],
                        out_hbm.at[c, pl.ds(s * rows_per_tile, rows_per_tile)])

    return msg_kernel(src_t, dst_t, h2b, zeros_tile)


def _tc_epilogue(parts, h2, dinv, b):
    n_pad, dout = h2.shape
    blk = 2048
    grid = n_pad // blk

    def body(p_ref, h2_ref, dinv_ref, b_ref, o_ref):
        msgs = (p_ref[0].astype(jnp.float32) + p_ref[1].astype(jnp.float32))
        tot = msgs + h2_ref[...]
        dinv = dinv_ref[pl.ds(pl.program_id(0) * blk, blk)]
        o_ref[...] = jnp.maximum(
            tot * dinv[:, None] + b_ref[...][None, :], 0.0)

    return pl.pallas_call(
        body,
        grid=(grid,),
        in_specs=[
            pl.BlockSpec((NC, blk, dout), lambda i: (0, i, 0)),
            pl.BlockSpec((blk, dout), lambda i: (i, 0)),
            pl.BlockSpec((n_pad,), lambda i: (0,)),
            pl.BlockSpec((dout,), lambda i: (0,)),
        ],
        out_specs=pl.BlockSpec((blk, dout), lambda i: (i, 0)),
        out_shape=jax.ShapeDtypeStruct((n_pad, dout), jnp.float32),
    )(parts, h2, dinv, b)


def kernel(x, edge_index, W, b):
    n, din = x.shape
    dout = W.shape[1]
    e = edge_index.shape[1]
    src = edge_index[0].astype(jnp.int32)
    dst = edge_index[1].astype(jnp.int32)

    ck = 128
    n_pad = ((n + NS * ck - 1) // (NS * ck)) * (NS * ck)
    x_pad = jnp.concatenate(
        [x, jnp.zeros((n_pad - n, din), jnp.float32)], axis=0)

    e_deg = ((e + NW * LANES - 1) // (NW * LANES)) * (NW * LANES)
    dst_deg = jnp.concatenate(
        [dst, jnp.full((e_deg - e,), n, jnp.int32)])
    zeros_hist = jnp.zeros((n_pad,), jnp.float32)
    deg_parts = _sc_degree(dst_deg, zeros_hist, n_pad)

    h2, h2b, dinv = _tc_transform(x_pad, W, deg_parts)

    e_pad = ((e + NW * ck * NBUF - 1) // (NW * ck * NBUF)) * (NW * ck * NBUF)
    pad = jnp.full((e_pad - e,), n, jnp.int32)
    src_t = jnp.concatenate([src, pad]).reshape(NW, e_pad // (NW * ck), ck)
    dst_t = jnp.concatenate([dst, pad]).reshape(NW, e_pad // (NW * ck), ck)
    zeros_tile = jnp.zeros((ck, dout), jnp.bfloat16)
    parts = _sc_messages(src_t, dst_t, h2b, zeros_tile)

    return _tc_epilogue(parts, h2, dinv, b)[:n]


# speedup vs baseline: 1.9479x; 1.7578x over previous
import dataclasses
import functools

import jax
import jax.numpy as jnp
from jax import lax
from jax.experimental import pallas as pl
from jax.experimental.pallas import tpu as pltpu
from jax.experimental.pallas import tpu_sc as plsc

NC = 2
NS = 16
NW = NC * NS
LANES = 16
NBUF = 3


def _sc_compiler_params(tc_tiling=True):
    cp = pltpu.CompilerParams()
    if "needs_layout_passes" in pltpu.CompilerParams.__dataclass_fields__:
        cp = dataclasses.replace(cp, needs_layout_passes=False)
    if not tc_tiling:
        cp = dataclasses.replace(cp, use_tc_tiling_on_sc=False)
    return cp


def _sc_degree(dst, zeros_hist, n_pad):
    ep = dst.shape[0]
    e_per_tile = ep // NW
    mesh = plsc.VectorSubcoreMesh(core_axis_name="c", subcore_axis_name="s")

    @functools.partial(
        pl.kernel,
        out_type=jax.ShapeDtypeStruct((NW, n_pad), jnp.float32),
        mesh=mesh,
        scratch_types=[
            pltpu.VMEM((e_per_tile,), jnp.int32),
            pltpu.VMEM((n_pad,), jnp.float32),
        ],
        compiler_params=_sc_compiler_params(),
    )
    def deg_kernel(dst_hbm, zeros_hbm, out_hbm, dst_v, hist):
        c = lax.axis_index("c")
        s = lax.axis_index("s")
        wid = c * NS + s
        pltpu.sync_copy(dst_hbm.at[pl.ds(wid * e_per_tile, e_per_tile)], dst_v)
        pltpu.sync_copy(zeros_hbm, hist)
        ones = jnp.ones((LANES,), jnp.float32)

        @pl.loop(0, e_per_tile // LANES)
        def _(i):
            d = dst_v[pl.ds(i * LANES, LANES)]
            plsc.addupdate_scatter(hist, [d], ones)

        pltpu.sync_copy(hist, out_hbm.at[wid])

    return deg_kernel(dst, zeros_hist)


def _tc_transform(x_pad, W, deg_parts):
    n_pad, din = x_pad.shape
    dout = W.shape[1]
    blk = 1280
    grid = n_pad // blk

    def body(x_ref, w_ref, deg_ref, h2_ref, h2b_ref, dinv_ref):
        deg = jnp.sum(deg_ref[...], axis=0) + 1.0
        dinv = lax.rsqrt(deg)
        h = jnp.dot(x_ref[...], w_ref[...], preferred_element_type=jnp.float32)
        h2 = h * dinv[:, None]
        h2_ref[...] = h2
        h2b_ref[...] = h2.astype(jnp.bfloat16)
        dinv_ref[pl.ds(pl.program_id(0) * blk, blk)] = dinv

    return pl.pallas_call(
        body,
        grid=(grid,),
        in_specs=[
            pl.BlockSpec((blk, din), lambda i: (i, 0)),
            pl.BlockSpec((din, dout), lambda i: (0, 0)),
            pl.BlockSpec((NW, blk), lambda i: (0, i)),
        ],
        out_specs=[
            pl.BlockSpec((blk, dout), lambda i: (i, 0)),
            pl.BlockSpec((blk, dout), lambda i: (i, 0)),
            pl.BlockSpec((n_pad,), lambda i: (0,)),
        ],
        out_shape=[
            jax.ShapeDtypeStruct((n_pad, dout), jnp.float32),
            jax.ShapeDtypeStruct((n_pad, dout), jnp.bfloat16),
            jax.ShapeDtypeStruct((n_pad,), jnp.float32),
        ],
    )(x_pad, W, deg_parts)


def _sc_messages(src_t, dst_t, h2b, zeros_tile):
    _, ch, ck = src_t.shape
    n_pad, dout = h2b.shape
    rows_per_tile = n_pad // NS
    n_zero_copies = rows_per_tile // ck
    rounds = ch // NBUF
    mesh = plsc.VectorSubcoreMesh(core_axis_name="c", subcore_axis_name="s")

    @functools.partial(
        pl.kernel,
        out_type=jax.ShapeDtypeStruct((NC, n_pad, dout), jnp.bfloat16),
        mesh=mesh,
        scratch_types=(
            [pltpu.VMEM((ch, ck), jnp.int32),
             pltpu.VMEM((ch, ck), jnp.int32),
             pltpu.VMEM_SHARED((n_pad, dout), jnp.bfloat16),
             pltpu.VMEM_SHARED((n_pad, dout), jnp.bfloat16)]
            + [pltpu.VMEM((ck, dout), jnp.bfloat16) for _ in range(NBUF)]
            + [pltpu.SemaphoreType.DMA for _ in range(2 * NBUF)]
        ),
        compiler_params=_sc_compiler_params(tc_tiling=False),
    )
    def msg_kernel(src_hbm, dst_hbm, h2_hbm, z_hbm, out_hbm,
                   src_v, dst_v, acc, table, *bufs_and_sems):
        bufs = bufs_and_sems[:NBUF]
        gsems = bufs_and_sems[NBUF:2 * NBUF]
        ssems = bufs_and_sems[2 * NBUF:3 * NBUF]
        c = lax.axis_index("c")
        s = lax.axis_index("s")
        wid = c * NS + s
        pltpu.sync_copy(src_hbm.at[wid], src_v)
        pltpu.sync_copy(dst_hbm.at[wid], dst_v)
        pltpu.sync_copy(h2_hbm.at[pl.ds(s * rows_per_tile, rows_per_tile)],
                        table.at[pl.ds(s * rows_per_tile, rows_per_tile)])
        pltpu.sync_copy(z_hbm, bufs[0])
        for k in range(n_zero_copies):
            pltpu.sync_copy(bufs[0], acc.at[pl.ds(s * rows_per_tile + k * ck, ck)])
        plsc.subcore_barrier()

        def start_gather(j, b):
            pltpu.async_copy(table.at[src_v.at[j]], bufs[b], gsems[b])

        def wait_gather(j, b):
            pltpu.make_async_copy(table.at[src_v.at[j]], bufs[b],
                                  gsems[b]).wait()

        def start_scatter(j, b):
            pltpu.async_copy(bufs[b], acc.at[dst_v.at[j]], ssems[b], add=True)

        def wait_scatter(j, b):
            pltpu.make_async_copy(bufs[b], acc.at[dst_v.at[j]],
                                  ssems[b]).wait()

        for b in range(NBUF):
            start_gather(b, b)

        @pl.loop(0, rounds - 1)
        def _(jj):
            j0 = jj * NBUF
            for b in range(NBUF):
                wait_gather(j0 + b, b)
                start_scatter(j0 + b, b)
            for b in range(NBUF):
                wait_scatter(j0 + b, b)
                start_gather(j0 + NBUF + b, b)

        j0 = (rounds - 1) * NBUF
        for b in range(NBUF):
            wait_gather(j0 + b, b)
            start_scatter(j0 + b, b)
        for b in range(NBUF):
            wait_scatter(j0 + b, b)

        plsc.subcore_barrier()
        pltpu.sync_copy(acc.at[pl.ds(s * rows_per_tile, rows_per_tile)],
                        out_hbm.at[c, pl.ds(s * rows_per_tile, rows_per_tile)])

    return msg_kernel(src_t, dst_t, h2b, zeros_tile)


def _tc_epilogue(parts, h2, dinv, b):
    n_pad, dout = h2.shape
    blk = 2048
    grid = n_pad // blk

    def body(p_ref, h2_ref, dinv_ref, b_ref, o_ref):
        msgs = (p_ref[0].astype(jnp.float32) + p_ref[1].astype(jnp.float32))
        tot = msgs + h2_ref[...]
        dinv = dinv_ref[pl.ds(pl.program_id(0) * blk, blk)]
        o_ref[...] = jnp.maximum(
            tot * dinv[:, None] + b_ref[...][None, :], 0.0)

    return pl.pallas_call(
        body,
        grid=(grid,),
        in_specs=[
            pl.BlockSpec((NC, blk, dout), lambda i: (0, i, 0)),
            pl.BlockSpec((blk, dout), lambda i: (i, 0)),
            pl.BlockSpec((n_pad,), lambda i: (0,)),
            pl.BlockSpec((dout,), lambda i: (0,)),
        ],
        out_specs=pl.BlockSpec((blk, dout), lambda i: (i, 0)),
        out_shape=jax.ShapeDtypeStruct((n_pad, dout), jnp.float32),
    )(parts, h2, dinv, b)


def kernel(x, edge_index, W, b):
    n, din = x.shape
    dout = W.shape[1]
    e = edge_index.shape[1]
    src = edge_index[0].astype(jnp.int32)
    dst = edge_index[1].astype(jnp.int32)

    ck = 128
    n_pad = ((n + NS * ck - 1) // (NS * ck)) * (NS * ck)
    x_pad = jnp.concatenate(
        [x, jnp.zeros((n_pad - n, din), jnp.float32)], axis=0)

    e_deg = ((e + NW * LANES - 1) // (NW * LANES)) * (NW * LANES)
    dst_deg = jnp.concatenate(
        [dst, jnp.full((e_deg - e,), n, jnp.int32)])
    zeros_hist = jnp.zeros((n_pad,), jnp.float32)
    deg_parts = _sc_degree(dst_deg, zeros_hist, n_pad)

    h2, h2b, dinv = _tc_transform(x_pad, W, deg_parts)

    e_pad = ((e + NW * ck * NBUF - 1) // (NW * ck * NBUF)) * (NW * ck * NBUF)
    pad = jnp.full((e_pad - e,), n, jnp.int32)
    src_t = jnp.concatenate([src, pad]).reshape(NW, e_pad // (NW * ck), ck)
    dst_t = jnp.concatenate([dst, pad]).reshape(NW, e_pad // (NW * ck), ck)
    zeros_tile = jnp.zeros((ck, dout), jnp.bfloat16)
    parts = _sc_messages(src_t, dst_t, h2b, zeros_tile)

    return _tc_epilogue(parts, h2, dinv, b)[:n]


# speedup vs baseline: 1.9726x; 1.0127x over previous
import dataclasses
import functools

import jax
import jax.numpy as jnp
from jax import lax
from jax.experimental import pallas as pl
from jax.experimental.pallas import tpu as pltpu
from jax.experimental.pallas import tpu_sc as plsc

NC = 2
NS = 16
NW = NC * NS
LANES = 16
NBUF = 3


def _sc_compiler_params(tc_tiling=True):
    cp = pltpu.CompilerParams()
    if "needs_layout_passes" in pltpu.CompilerParams.__dataclass_fields__:
        cp = dataclasses.replace(cp, needs_layout_passes=False)
    if not tc_tiling:
        cp = dataclasses.replace(cp, use_tc_tiling_on_sc=False)
    return cp


def _sc_degree(ei_flat, zeros_hist, n_pad):
    e_per_tile = ei_flat.shape[2]
    mesh = plsc.VectorSubcoreMesh(core_axis_name="c", subcore_axis_name="s")

    @functools.partial(
        pl.kernel,
        out_type=jax.ShapeDtypeStruct((NW, n_pad), jnp.float32),
        mesh=mesh,
        scratch_types=[
            pltpu.VMEM((e_per_tile,), jnp.int32),
            pltpu.VMEM((n_pad,), jnp.float32),
        ],
        compiler_params=_sc_compiler_params(),
    )
    def deg_kernel(ei_hbm, zeros_hbm, out_hbm, dst_v, hist):
        c = lax.axis_index("c")
        s = lax.axis_index("s")
        wid = c * NS + s
        pltpu.sync_copy(ei_hbm.at[1, wid], dst_v)
        pltpu.sync_copy(zeros_hbm, hist)
        ones = jnp.ones((LANES,), jnp.float32)

        @pl.loop(0, e_per_tile // LANES)
        def _(i):
            d = dst_v[pl.ds(i * LANES, LANES)]
            plsc.addupdate_scatter(hist, [d], ones)

        pltpu.sync_copy(hist, out_hbm.at[wid])

    return deg_kernel(ei_flat, zeros_hist)


def _tc_transform(x_pad, W, deg_parts):
    n_pad, din = x_pad.shape
    dout = W.shape[1]
    blk = 1280
    grid = n_pad // blk

    def body(x_ref, w_ref, deg_ref, h2_ref, h2b_ref, dinv_ref):
        deg = jnp.sum(deg_ref[...], axis=0) + 1.0
        dinv = lax.rsqrt(deg)
        h = jnp.dot(x_ref[...], w_ref[...], preferred_element_type=jnp.float32)
        h2 = h * dinv[:, None]
        h2_ref[...] = h2
        h2b_ref[...] = h2.astype(jnp.bfloat16)
        dinv_ref[pl.ds(pl.program_id(0) * blk, blk)] = dinv

    return pl.pallas_call(
        body,
        grid=(grid,),
        in_specs=[
            pl.BlockSpec((blk, din), lambda i: (i, 0)),
            pl.BlockSpec((din, dout), lambda i: (0, 0)),
            pl.BlockSpec((NW, blk), lambda i: (0, i)),
        ],
        out_specs=[
            pl.BlockSpec((blk, dout), lambda i: (i, 0)),
            pl.BlockSpec((blk, dout), lambda i: (i, 0)),
            pl.BlockSpec((n_pad,), lambda i: (0,)),
        ],
        out_shape=[
            jax.ShapeDtypeStruct((n_pad, dout), jnp.float32),
            jax.ShapeDtypeStruct((n_pad, dout), jnp.bfloat16),
            jax.ShapeDtypeStruct((n_pad,), jnp.float32),
        ],
    )(x_pad, W, deg_parts)


def _sc_messages(ei_t, h2b, zeros_tile):
    _, _, ch, ck = ei_t.shape
    n_pad, dout = h2b.shape
    rows_per_tile = n_pad // NS
    n_zero_copies = rows_per_tile // ck
    rounds = ch // NBUF
    mesh = plsc.VectorSubcoreMesh(core_axis_name="c", subcore_axis_name="s")

    @functools.partial(
        pl.kernel,
        out_type=jax.ShapeDtypeStruct((NC, n_pad, dout), jnp.bfloat16),
        mesh=mesh,
        scratch_types=(
            [pltpu.VMEM((ch, ck), jnp.int32),
             pltpu.VMEM((ch, ck), jnp.int32),
             pltpu.VMEM_SHARED((n_pad, dout), jnp.bfloat16),
             pltpu.VMEM_SHARED((n_pad, dout), jnp.bfloat16)]
            + [pltpu.VMEM((ck, dout), jnp.bfloat16) for _ in range(NBUF)]
            + [pltpu.SemaphoreType.DMA for _ in range(2 * NBUF)]
        ),
        compiler_params=_sc_compiler_params(tc_tiling=False),
    )
    def msg_kernel(ei_hbm, h2_hbm, z_hbm, out_hbm,
                   src_v, dst_v, acc, table, *bufs_and_sems):
        bufs = bufs_and_sems[:NBUF]
        gsems = bufs_and_sems[NBUF:2 * NBUF]
        ssems = bufs_and_sems[2 * NBUF:3 * NBUF]
        c = lax.axis_index("c")
        s = lax.axis_index("s")
        wid = c * NS + s
        pltpu.sync_copy(ei_hbm.at[0, wid], src_v)
        pltpu.sync_copy(ei_hbm.at[1, wid], dst_v)
        pltpu.sync_copy(h2_hbm.at[pl.ds(s * rows_per_tile, rows_per_tile)],
                        table.at[pl.ds(s * rows_per_tile, rows_per_tile)])
        pltpu.sync_copy(z_hbm, bufs[0])
        for k in range(n_zero_copies):
            pltpu.sync_copy(bufs[0], acc.at[pl.ds(s * rows_per_tile + k * ck, ck)])
        plsc.subcore_barrier()

        def start_gather(j, b):
            pltpu.async_copy(table.at[src_v.at[j]], bufs[b], gsems[b])

        def wait_gather(j, b):
            pltpu.make_async_copy(table.at[src_v.at[j]], bufs[b],
                                  gsems[b]).wait()

        def start_scatter(j, b):
            pltpu.async_copy(bufs[b], acc.at[dst_v.at[j]], ssems[b], add=True)

        def wait_scatter(j, b):
            pltpu.make_async_copy(bufs[b], acc.at[dst_v.at[j]],
                                  ssems[b]).wait()

        for b in range(NBUF):
            start_gather(b, b)

        @pl.loop(0, rounds - 1)
        def _(jj):
            j0 = jj * NBUF
            for b in range(NBUF):
                wait_gather(j0 + b, b)
                start_scatter(j0 + b, b)
            for b in range(NBUF):
                wait_scatter(j0 + b, b)
                start_gather(j0 + NBUF + b, b)

        j0 = (rounds - 1) * NBUF
        for b in range(NBUF):
            wait_gather(j0 + b, b)
            start_scatter(j0 + b, b)
        for b in range(NBUF):
            wait_scatter(j0 + b, b)

        plsc.subcore_barrier()
        pltpu.sync_copy(acc.at[pl.ds(s * rows_per_tile, rows_per_tile)],
                        out_hbm.at[c, pl.ds(s * rows_per_tile, rows_per_tile)])

    return msg_kernel(ei_t, h2b, zeros_tile)


def _tc_epilogue(parts, h2, dinv, b):
    n_pad, dout = h2.shape
    blk = 2048
    grid = n_pad // blk

    def body(p_ref, h2_ref, dinv_ref, b_ref, o_ref):
        msgs = (p_ref[0].astype(jnp.float32) + p_ref[1].astype(jnp.float32))
        tot = msgs + h2_ref[...]
        dinv = dinv_ref[pl.ds(pl.program_id(0) * blk, blk)]
        o_ref[...] = jnp.maximum(
            tot * dinv[:, None] + b_ref[...][None, :], 0.0)

    return pl.pallas_call(
        body,
        grid=(grid,),
        in_specs=[
            pl.BlockSpec((NC, blk, dout), lambda i: (0, i, 0)),
            pl.BlockSpec((blk, dout), lambda i: (i, 0)),
            pl.BlockSpec((n_pad,), lambda i: (0,)),
            pl.BlockSpec((dout,), lambda i: (0,)),
        ],
        out_specs=pl.BlockSpec((blk, dout), lambda i: (i, 0)),
        out_shape=jax.ShapeDtypeStruct((n_pad, dout), jnp.float32),
    )(parts, h2, dinv, b)


def kernel(x, edge_index, W, b):
    n, din = x.shape
    dout = W.shape[1]
    e = edge_index.shape[1]

    ck = 128
    n_pad = ((n + NS * ck - 1) // (NS * ck)) * (NS * ck)
    x_pad = jnp.concatenate(
        [x, jnp.zeros((n_pad - n, din), jnp.float32)], axis=0)

    e_pad = ((e + NW * ck * NBUF - 1) // (NW * ck * NBUF)) * (NW * ck * NBUF)
    ei = jnp.concatenate(
        [edge_index.astype(jnp.int32),
         jnp.full((2, e_pad - e), n, jnp.int32)], axis=1)
    ei_t = ei.reshape(2, NW, e_pad // (NW * ck), ck)
    ei_flat = ei.reshape(2, NW, e_pad // NW)

    zeros_hist = jnp.zeros((n_pad,), jnp.float32)
    deg_parts = _sc_degree(ei_flat, zeros_hist, n_pad)

    h2, h2b, dinv = _tc_transform(x_pad, W, deg_parts)

    zeros_tile = jnp.zeros((ck, dout), jnp.bfloat16)
    parts = _sc_messages(ei_t, h2b, zeros_tile)

    return _tc_epilogue(parts, h2, dinv, b)[:n]


# speedup vs baseline: 2.0654x; 1.0470x over previous
import dataclasses
import functools

import jax
import jax.numpy as jnp
from jax import lax
from jax.experimental import pallas as pl
from jax.experimental.pallas import tpu as pltpu
from jax.experimental.pallas import tpu_sc as plsc

NC = 2
NS = 16
NW = NC * NS
LANES = 16
NBUF = 3


def _sc_compiler_params(tc_tiling=True):
    cp = pltpu.CompilerParams()
    if "needs_layout_passes" in pltpu.CompilerParams.__dataclass_fields__:
        cp = dataclasses.replace(cp, needs_layout_passes=False)
    if not tc_tiling:
        cp = dataclasses.replace(cp, use_tc_tiling_on_sc=False)
    return cp


def _sc_degree(ei_flat, zeros_hist, n_pad):
    e_per_tile = ei_flat.shape[2]
    mesh = plsc.VectorSubcoreMesh(core_axis_name="c", subcore_axis_name="s")

    @functools.partial(
        pl.kernel,
        out_type=jax.ShapeDtypeStruct((NW, n_pad), jnp.float32),
        mesh=mesh,
        scratch_types=[
            pltpu.VMEM((e_per_tile,), jnp.int32),
            pltpu.VMEM((n_pad,), jnp.float32),
        ],
        compiler_params=_sc_compiler_params(),
    )
    def deg_kernel(ei_hbm, zeros_hbm, out_hbm, dst_v, hist):
        c = lax.axis_index("c")
        s = lax.axis_index("s")
        wid = c * NS + s
        pltpu.sync_copy(ei_hbm.at[1, wid], dst_v)
        pltpu.sync_copy(zeros_hbm, hist)
        ones = jnp.ones((LANES,), jnp.float32)

        @pl.loop(0, e_per_tile // LANES)
        def _(i):
            d = dst_v[pl.ds(i * LANES, LANES)]
            plsc.addupdate_scatter(hist, [d], ones)

        pltpu.sync_copy(hist, out_hbm.at[wid])

    return deg_kernel(ei_flat, zeros_hist)


def _tc_transform(x_pad, W, deg_parts):
    n_pad, din = x_pad.shape
    dout = W.shape[1]
    blk = 1280
    grid = n_pad // blk

    def body(x_ref, w_ref, deg_ref, h2b_ref, dinv_ref):
        deg = jnp.sum(deg_ref[...], axis=0) + 1.0
        dinv = lax.rsqrt(deg)
        h = jnp.dot(x_ref[...], w_ref[...], preferred_element_type=jnp.float32)
        h2 = h * dinv[:, None]
        h2b_ref[...] = h2.astype(jnp.bfloat16)
        dinv_ref[...] = jnp.broadcast_to(
            dinv.astype(jnp.bfloat16)[:, None], (blk, 32))

    return pl.pallas_call(
        body,
        grid=(grid,),
        in_specs=[
            pl.BlockSpec((blk, din), lambda i: (i, 0)),
            pl.BlockSpec((din, dout), lambda i: (0, 0)),
            pl.BlockSpec((NW, blk), lambda i: (0, i)),
        ],
        out_specs=[
            pl.BlockSpec((blk, dout), lambda i: (i, 0)),
            pl.BlockSpec((blk, 32), lambda i: (i, 0)),
        ],
        out_shape=[
            jax.ShapeDtypeStruct((n_pad, dout), jnp.bfloat16),
            jax.ShapeDtypeStruct((n_pad, 32), jnp.bfloat16),
        ],
    )(x_pad, W, deg_parts)


def _sc_messages(ei_t, h2b, zeros_tile):
    _, _, ch, ck = ei_t.shape
    n_pad, dout = h2b.shape
    rows_per_tile = n_pad // NS
    n_zero_copies = rows_per_tile // ck
    rounds = ch // NBUF
    mesh = plsc.VectorSubcoreMesh(core_axis_name="c", subcore_axis_name="s")

    @functools.partial(
        pl.kernel,
        out_type=jax.ShapeDtypeStruct((NC, n_pad, dout), jnp.bfloat16),
        mesh=mesh,
        scratch_types=(
            [pltpu.VMEM((ch, ck), jnp.int32),
             pltpu.VMEM((ch, ck), jnp.int32),
             pltpu.VMEM_SHARED((n_pad, dout), jnp.bfloat16),
             pltpu.VMEM_SHARED((n_pad, dout), jnp.bfloat16)]
            + [pltpu.VMEM((ck, dout), jnp.bfloat16) for _ in range(NBUF)]
            + [pltpu.SemaphoreType.DMA for _ in range(2 * NBUF)]
        ),
        compiler_params=_sc_compiler_params(tc_tiling=False),
    )
    def msg_kernel(ei_hbm, h2_hbm, z_hbm, out_hbm,
                   src_v, dst_v, acc, table, *bufs_and_sems):
        bufs = bufs_and_sems[:NBUF]
        gsems = bufs_and_sems[NBUF:2 * NBUF]
        ssems = bufs_and_sems[2 * NBUF:3 * NBUF]
        c = lax.axis_index("c")
        s = lax.axis_index("s")
        wid = c * NS + s
        pltpu.sync_copy(ei_hbm.at[0, wid], src_v)
        pltpu.sync_copy(ei_hbm.at[1, wid], dst_v)
        pltpu.sync_copy(h2_hbm.at[pl.ds(s * rows_per_tile, rows_per_tile)],
                        table.at[pl.ds(s * rows_per_tile, rows_per_tile)])
        @pl.when(c == 0)
        def _():
            pltpu.sync_copy(
                h2_hbm.at[pl.ds(s * rows_per_tile, rows_per_tile)],
                acc.at[pl.ds(s * rows_per_tile, rows_per_tile)])

        @pl.when(c != 0)
        def _():
            pltpu.sync_copy(z_hbm, bufs[0])
            for k in range(n_zero_copies):
                pltpu.sync_copy(
                    bufs[0], acc.at[pl.ds(s * rows_per_tile + k * ck, ck)])
        plsc.subcore_barrier()

        def start_gather(j, b):
            pltpu.async_copy(table.at[src_v.at[j]], bufs[b], gsems[b])

        def wait_gather(j, b):
            pltpu.make_async_copy(table.at[src_v.at[j]], bufs[b],
                                  gsems[b]).wait()

        def start_scatter(j, b):
            pltpu.async_copy(bufs[b], acc.at[dst_v.at[j]], ssems[b], add=True)

        def wait_scatter(j, b):
            pltpu.make_async_copy(bufs[b], acc.at[dst_v.at[j]],
                                  ssems[b]).wait()

        for b in range(NBUF):
            start_gather(b, b)

        @pl.loop(0, rounds - 1)
        def _(jj):
            j0 = jj * NBUF
            for b in range(NBUF):
                wait_gather(j0 + b, b)
                start_scatter(j0 + b, b)
            for b in range(NBUF):
                wait_scatter(j0 + b, b)
                start_gather(j0 + NBUF + b, b)

        j0 = (rounds - 1) * NBUF
        for b in range(NBUF):
            wait_gather(j0 + b, b)
            start_scatter(j0 + b, b)
        for b in range(NBUF):
            wait_scatter(j0 + b, b)

        plsc.subcore_barrier()
        pltpu.sync_copy(acc.at[pl.ds(s * rows_per_tile, rows_per_tile)],
                        out_hbm.at[c, pl.ds(s * rows_per_tile, rows_per_tile)])

    return msg_kernel(ei_t, h2b, zeros_tile)


def _sc_epilogue(parts, dinv, b_bf, n_out):
    _, n_pad, dout = parts.shape
    rows_per_w = n_pad // NW
    full_w = n_out // rows_per_w
    rem = n_out - full_w * rows_per_w
    rb = 128
    groups = dout // 32
    mesh = plsc.VectorSubcoreMesh(core_axis_name="c", subcore_axis_name="s")

    def blocks_of(total):
        out = []
        o = 0
        while o < total:
            out.append((o, min(rb, total - o)))
            o += rb
        return out

    @functools.partial(
        pl.kernel,
        out_type=jax.ShapeDtypeStruct((n_out, dout), jnp.bfloat16),
        mesh=mesh,
        scratch_types=[
            pltpu.VMEM((rb, dout), jnp.bfloat16),
            pltpu.VMEM((rb, dout), jnp.bfloat16),
            pltpu.VMEM((rows_per_w, 32), jnp.bfloat16),
            pltpu.VMEM((dout,), jnp.bfloat16),
        ],
        compiler_params=_sc_compiler_params(tc_tiling=False),
    )
    def epi_kernel(p_hbm, dinv_hbm, b_hbm, out_hbm, p0v, p1v, dinv_v, bvec):
        c = lax.axis_index("c")
        s = lax.axis_index("s")
        wid = c * NS + s
        base = wid * rows_per_w
        pltpu.sync_copy(dinv_hbm.at[pl.ds(base, rows_per_w)], dinv_v)
        pltpu.sync_copy(b_hbm, bvec)
        bvals = [bvec[pl.ds(g * 32, 32)] for g in range(groups)]

        def do_block(off, nrows):
            pltpu.sync_copy(p_hbm.at[0, pl.ds(base + off, nrows)],
                            p0v.at[pl.ds(0, nrows)])
            pltpu.sync_copy(p_hbm.at[1, pl.ds(base + off, nrows)],
                            p1v.at[pl.ds(0, nrows)])

            @pl.loop(0, nrows)
            def _(r):
                dv = dinv_v.at[off + r][pl.ds(0, 32)]
                row0 = p0v.at[r]
                row1 = p1v.at[r]
                for g in range(groups):
                    sl = pl.ds(g * 32, 32)
                    v = row0[sl] + row1[sl]
                    row0[sl] = jnp.maximum(v * dv + bvals[g], 0)

            pltpu.sync_copy(p0v.at[pl.ds(0, nrows)],
                            out_hbm.at[pl.ds(base + off, nrows)])

        @pl.when(wid < full_w)
        def _():
            for off, nrows in blocks_of(rows_per_w):
                do_block(off, nrows)

        if rem:
            @pl.when(wid == full_w)
            def _():
                for off, nrows in blocks_of(rem):
                    do_block(off, nrows)

    return epi_kernel(parts, dinv, b_bf)


def kernel(x, edge_index, W, b):
    n, din = x.shape
    dout = W.shape[1]
    e = edge_index.shape[1]

    ck = 128
    n_pad = ((n + NS * ck - 1) // (NS * ck)) * (NS * ck)
    x_pad = jnp.concatenate(
        [x, jnp.zeros((n_pad - n, din), jnp.float32)], axis=0)

    e_pad = ((e + NW * ck * NBUF - 1) // (NW * ck * NBUF)) * (NW * ck * NBUF)
    ei = jnp.concatenate(
        [edge_index.astype(jnp.int32),
         jnp.full((2, e_pad - e), n, jnp.int32)], axis=1)
    ei_t = ei.reshape(2, NW, e_pad // (NW * ck), ck)
    ei_flat = ei.reshape(2, NW, e_pad // NW)

    zeros_hist = jnp.zeros((n_pad,), jnp.float32)
    deg_parts = _sc_degree(ei_flat, zeros_hist, n_pad)

    h2b, dinv = _tc_transform(x_pad, W, deg_parts)

    zeros_tile = jnp.zeros((ck, dout), jnp.bfloat16)
    parts = _sc_messages(ei_t, h2b, zeros_tile)

    out_bf = _sc_epilogue(parts, dinv, b.astype(jnp.bfloat16), n)
    return out_bf.astype(jnp.float32)
